# Initial kernel scaffold; baseline (speedup 1.0000x reference)
import jax
import jax.numpy as jnp
from jax.experimental import pallas as pl


def kernel(visit_segments, embedding_table):
    raise NotImplementedError("write your pallas kernel here")


# speedup vs baseline: 3.4221x; 3.4221x over previous
import functools

import jax
import jax.numpy as jnp
from jax import lax
from jax.experimental import pallas as pl
from jax.experimental.pallas import tpu as pltpu
from jax.experimental.pallas import tpu_sc as plsc

_B = 4096
_L = 200
_D = 64
_N = _B * _L
_G = 128
_info = plsc.get_sparse_core_info()
_NC = _info.num_cores
_NS = _info.num_subcores
_NW = _NC * _NS
_PER_W = _N // _NW
_K = _PER_W // _G

_mesh = plsc.VectorSubcoreMesh(core_axis_name="c", subcore_axis_name="s")


@functools.partial(
    pl.kernel,
    mesh=_mesh,
    out_type=jax.ShapeDtypeStruct((_N, _D), jnp.float32),
    scratch_types=[
        pltpu.VMEM((_K, _G), jnp.int32),
        pltpu.VMEM((_G, _D), jnp.float32),
        pltpu.SemaphoreType.DMA,
    ],
    compiler_params=pltpu.CompilerParams(use_tc_tiling_on_sc=False),
)
def _sc_gather(idx_hbm, table_hbm, out_hbm, idx_v, rows_v, sem):
    wid = lax.axis_index("s") * _NC + lax.axis_index("c")
    base = wid * _K
    pltpu.sync_copy(idx_hbm.at[pl.ds(base, _K)], idx_v)

    def step(j, carry):
        pltpu.async_copy(table_hbm.at[idx_v.at[j]], rows_v, sem).wait()
        pltpu.sync_copy(rows_v, out_hbm.at[pl.ds((base + j) * _G, _G)])
        return carry

    lax.fori_loop(0, _K, step, 0)


def kernel(visit_segments, embedding_table):
    idx = visit_segments.reshape(_N // _G, _G)
    out = _sc_gather(idx, embedding_table)
    return out.reshape(_B, _L, _D)


# speedup vs baseline: 3.5901x; 1.0491x over previous
import functools

import jax
import jax.numpy as jnp
from jax import lax
from jax.experimental import pallas as pl
from jax.experimental.pallas import tpu as pltpu
from jax.experimental.pallas import tpu_sc as plsc

_B = 4096
_L = 200
_D = 64
_N = _B * _L
_G = 128
_info = plsc.get_sparse_core_info()
_NC = _info.num_cores
_NS = _info.num_subcores
_NW = _NC * _NS
_PER_W = _N // _NW
_K = _PER_W // _G

_mesh = plsc.VectorSubcoreMesh(core_axis_name="c", subcore_axis_name="s")


_NBUF = 4
_NGRP = _K // _NBUF


@functools.partial(
    pl.kernel,
    mesh=_mesh,
    out_type=jax.ShapeDtypeStruct((_N, _D), jnp.float32),
    scratch_types=[
        pltpu.VMEM((_K, _G), jnp.int32),
        pltpu.VMEM((_NBUF, _G, _D), jnp.float32),
        pltpu.SemaphoreType.DMA((_NBUF,)),
        pltpu.SemaphoreType.DMA((_NBUF,)),
    ],
    compiler_params=pltpu.CompilerParams(use_tc_tiling_on_sc=False),
)
def _sc_gather(idx_hbm, table_hbm, out_hbm, idx_v, rows_v, gsem, osem):
    wid = lax.axis_index("s") * _NC + lax.axis_index("c")
    base = wid * _K
    pltpu.sync_copy(idx_hbm.at[pl.ds(base, _K)], idx_v)

    def start_gather(j, b):
        pltpu.async_copy(table_hbm.at[idx_v.at[j]], rows_v.at[b], gsem.at[b])

    def wait_gather(j, b):
        pltpu.make_async_copy(
            table_hbm.at[idx_v.at[j]], rows_v.at[b], gsem.at[b]).wait()

    def start_scatter(j, b):
        pltpu.async_copy(
            rows_v.at[b], out_hbm.at[pl.ds((base + j) * _G, _G)], osem.at[b])

    def wait_scatter(j, b):
        pltpu.make_async_copy(
            rows_v.at[b], out_hbm.at[pl.ds((base + j) * _G, _G)],
            osem.at[b]).wait()

    for b in range(_NBUF):
        start_gather(b, b)

    def group(g, carry):
        for b in range(_NBUF):
            j = g * _NBUF + b
            wait_gather(j, b)
            start_scatter(j, b)
        for b in range(_NBUF):
            j = g * _NBUF + b
            wait_scatter(j, b)
            start_gather(j + _NBUF, b)
        return carry

    lax.fori_loop(0, _NGRP - 1, group, 0)

    for b in range(_NBUF):
        j = (_NGRP - 1) * _NBUF + b
        wait_gather(j, b)
        start_scatter(j, b)
    for b in range(_NBUF):
        j = (_NGRP - 1) * _NBUF + b
        wait_scatter(j, b)


def kernel(visit_segments, embedding_table):
    idx = visit_segments.reshape(_N // _G, _G)
    out = _sc_gather(idx, embedding_table)
    return out.reshape(_B, _L, _D)


# speedup vs baseline: 5.0271x; 1.4002x over previous
import functools

import jax
import jax.numpy as jnp
from jax import lax
from jax.experimental import pallas as pl
from jax.experimental.pallas import tpu as pltpu
from jax.experimental.pallas import tpu_sc as plsc

_B = 4096
_L = 200
_D = 64
_N = _B * _L
_G = 128
_info = plsc.get_sparse_core_info()
_NC = _info.num_cores
_NS = _info.num_subcores
_NW = _NC * _NS
_PER_W = _N // _NW
_K = _PER_W // _G

_mesh = plsc.VectorSubcoreMesh(core_axis_name="c", subcore_axis_name="s")


_NBUF = 4
_NGRP = _K // _NBUF


@functools.partial(
    pl.kernel,
    mesh=_mesh,
    out_type=jax.ShapeDtypeStruct((_N, _D), jnp.float32),
    scratch_types=[
        pltpu.VMEM((_K, _G), jnp.int32),
        pltpu.VMEM((_NBUF, _G, _D), jnp.float32),
        pltpu.VMEM_SHARED((1000, _D), jnp.float32),
        pltpu.SemaphoreType.DMA((_NBUF,)),
        pltpu.SemaphoreType.DMA((_NBUF,)),
    ],
    compiler_params=pltpu.CompilerParams(use_tc_tiling_on_sc=False),
)
def _sc_gather(idx_hbm, table_hbm, out_hbm, idx_v, rows_v, tab_s, gsem, osem):
    sid = lax.axis_index("s")
    wid = sid * _NC + lax.axis_index("c")
    base = wid * _K

    @pl.when(sid == 0)
    def _():
        pltpu.sync_copy(table_hbm, tab_s)

    pltpu.sync_copy(idx_hbm.at[pl.ds(base, _K)], idx_v)
    plsc.subcore_barrier()

    def start_gather(j, b):
        pltpu.async_copy(tab_s.at[idx_v.at[j]], rows_v.at[b], gsem.at[b])

    def wait_gather(j, b):
        pltpu.make_async_copy(
            tab_s.at[idx_v.at[j]], rows_v.at[b], gsem.at[b]).wait()

    def start_scatter(j, b):
        pltpu.async_copy(
            rows_v.at[b], out_hbm.at[pl.ds((base + j) * _G, _G)], osem.at[b])

    def wait_scatter(j, b):
        pltpu.make_async_copy(
            rows_v.at[b], out_hbm.at[pl.ds((base + j) * _G, _G)],
            osem.at[b]).wait()

    for b in range(_NBUF):
        start_gather(b, b)

    def group(g, carry):
        for b in range(_NBUF):
            j = g * _NBUF + b
            wait_gather(j, b)
            start_scatter(j, b)
        for b in range(_NBUF):
            j = g * _NBUF + b
            wait_scatter(j, b)
            start_gather(j + _NBUF, b)
        return carry

    lax.fori_loop(0, _NGRP - 1, group, 0)

    for b in range(_NBUF):
        j = (_NGRP - 1) * _NBUF + b
        wait_gather(j, b)
        start_scatter(j, b)
    for b in range(_NBUF):
        j = (_NGRP - 1) * _NBUF + b
        wait_scatter(j, b)


def kernel(visit_segments, embedding_table):
    idx = visit_segments.reshape(_N // _G, _G)
    out = _sc_gather(idx, embedding_table)
    return out.reshape(_B, _L, _D)
